# baseline (speedup 1.0000x reference)
import functools

import jax
import jax.numpy as jnp
from jax import lax
from jax.experimental import pallas as pl
from jax.experimental.pallas import tpu as pltpu
from jax.experimental.pallas import tpu_sc as plsc

_CB = 2048
_SH = 11
_SEG_D = 336
_SEG_V = 560
_CH = 64
_NW = 32


def _softplus(x):
    return jnp.maximum(x, 0.0) + jnp.log1p(jnp.exp(-jnp.abs(x)))


def _dot(a, b):
    return jax.lax.dot(a.astype(jnp.bfloat16), b.astype(jnp.bfloat16),
                       preferred_element_type=jnp.float32)


def _qmlp_body(l2_ref, nz_ref, w0p, w0n, w0z, b0, w1, b1, w2, b2, q2_ref):
    pos = l2_ref[0]
    neg = l2_ref[1]
    h = _dot(pos, w0p[...]) + _dot(neg, w0n[...]) + _dot(nz_ref[...], w0z[...])
    h = jnp.maximum(h + b0[...], 0.0)
    h = jnp.maximum(_dot(h, w1[...]) + b1[...], 0.0)
    q = _dot(h, w2[...]) + b2[...]
    q2_ref[0] = _softplus(q)
    q2_ref[1] = _softplus(-q)


def _qmlp(l2, nz, q_params, T):
    W0, b0, W1, b1, W2, b2 = q_params
    Nv, H = l2.shape[1], l2.shape[2]
    P = nz.shape[1]
    w0p, w0n, w0z = W0[:H], W0[H:2 * H], W0[2 * H:]
    grid = Nv // T
    full = lambda shape: pl.BlockSpec(shape, lambda i: (0,) * len(shape))
    return pl.pallas_call(
        _qmlp_body,
        grid=(grid,),
        in_specs=[
            pl.BlockSpec((2, T, H), lambda i: (0, i, 0)),
            pl.BlockSpec((T, P), lambda i: (i, 0)),
            full(w0p.shape), full(w0n.shape), full(w0z.shape), full(b0.shape),
            full(W1.shape), full(b1.shape), full(W2.shape), full(b2.shape),
        ],
        out_specs=pl.BlockSpec((2, T, H), lambda i: (0, i, 0)),
        out_shape=jax.ShapeDtypeStruct((2, Nv, H), jnp.float32),
    )(l2, nz, w0p, w0n, w0z, b0, W1, b1, W2, b2)


def _vmlp_body(l2_ref, pc_ref, nc_ref, w0a, w0b, w0c, w0d, b0, w1, b1,
               w2a, w2b, b2a, b2b, o2_ref):
    h = (_dot(l2_ref[0], w0a[...]) + _dot(l2_ref[1], w0b[...])
         + _dot(pc_ref[...], w0c[...]) + _dot(nc_ref[...], w0d[...]))
    h = jnp.maximum(h + b0[...], 0.0)
    h = jnp.maximum(_dot(h, w1[...]) + b1[...], 0.0)
    o2_ref[0] = _dot(h, w2a[...]) + b2a[...]
    o2_ref[1] = _dot(h, w2b[...]) + b2b[...]


def _vmlp(l2, pc2l, nc2l, v_params, T):
    W0, b0, W1, b1, W2, b2 = v_params
    Nv, H = l2.shape[1], l2.shape[2]
    w0a, w0b, w0c, w0d = W0[:H], W0[H:2 * H], W0[2 * H:3 * H], W0[3 * H:]
    w2a, w2b = W2[:, :H], W2[:, H:]
    b2a, b2b = b2[:H], b2[H:]
    grid = Nv // T
    full = lambda shape: pl.BlockSpec(shape, lambda i: (0,) * len(shape))
    return pl.pallas_call(
        _vmlp_body,
        grid=(grid,),
        in_specs=[
            pl.BlockSpec((2, T, H), lambda i: (0, i, 0)),
            pl.BlockSpec((T, H), lambda i: (i, 0)),
            pl.BlockSpec((T, H), lambda i: (i, 0)),
            full(w0a.shape), full(w0b.shape), full(w0c.shape), full(w0d.shape),
            full(b0.shape), full(W1.shape), full(b1.shape),
            full(w2a.shape), full(w2b.shape), full(b2a.shape), full(b2b.shape),
        ],
        out_specs=pl.BlockSpec((2, T, H), lambda i: (0, i, 0)),
        out_shape=jax.ShapeDtypeStruct((2, Nv, H), jnp.float32),
    )(l2, pc2l, nc2l, w0a, w0b, w0c, w0d, b0, W1, b1, w2a, w2b, b2a, b2b)


def _cmlp_body(l2c_ref, ce_ref, ea_ref, w0a, w0b, w0c, b0, w1, b1, w2, b2,
               out_ref):
    e = jnp.exp(-ea_ref[...])
    h = (_dot(l2c_ref[...], w0a[...]) + _dot(ce_ref[...], w0b[...])
         + _dot(e, w0c[...]))
    h = jnp.maximum(h + b0[...], 0.0)
    h = jnp.maximum(_dot(h, w1[...]) + b1[...], 0.0)
    out_ref[...] = _dot(h, w2[...]) + b2[...]


def _cmlp(l2c, c_emb, e_arg, c_params, T):
    W0, b0, W1, b1, W2, b2 = c_params
    Nc, H = c_emb.shape
    w0a, w0b, w0c = W0[:H], W0[H:2 * H], W0[2 * H:]
    grid = Nc // T
    full = lambda shape: pl.BlockSpec(shape, lambda i: (0,) * len(shape))
    row = pl.BlockSpec((T, H), lambda i: (i, 0))
    return pl.pallas_call(
        _cmlp_body,
        grid=(grid,),
        in_specs=[row, row, row,
                  full(w0a.shape), full(w0b.shape), full(w0c.shape),
                  full(b0.shape), full(W1.shape), full(b1.shape),
                  full(W2.shape), full(b2.shape)],
        out_specs=row,
        out_shape=jax.ShapeDtypeStruct((Nc, H), jnp.float32),
    )(l2c, c_emb, e_arg, w0a, w0b, w0c, b0, W1, b1, W2, b2)


_MESH = dict(core_axis_name="c", subcore_axis_name="s")


def _wid():
    return lax.axis_index("s") * 2 + lax.axis_index("c")


def _sc_hist(pos_src, pos_dst, neg_src, neg_dst, nbd, nbv):
    E = pos_src.shape[0]
    chunk = (E // _NW) // 16 * 16
    last = E - (_NW - 1) * chunk
    nh = nbd + 2 * nbv
    lanes_hist = nh * 16

    @functools.partial(
        pl.kernel, mesh=plsc.VectorSubcoreMesh(**_MESH),
        compiler_params=pltpu.CompilerParams(needs_layout_passes=False),
        out_type=jax.ShapeDtypeStruct((_NW * lanes_hist,), jnp.int32),
        scratch_types=[pltpu.VMEM((last,), jnp.int32),
                       pltpu.VMEM((lanes_hist,), jnp.int32)])
    def k(ps, pd, ns, nd, cnt_out, ibuf, hist):
        w = _wid()
        base = w * chunk
        lanes = lax.iota(jnp.int32, 16)
        ones = jnp.ones((16,), jnp.int32)

        def zb(i, c):
            hist[pl.ds(i * 16, 16)] = jnp.zeros((16,), jnp.int32)
            return c
        lax.fori_loop(0, nh, zb, 0)

        def do_list(src_hbm, obase):
            @pl.when(w < _NW - 1)
            def _():
                pltpu.sync_copy(src_hbm.at[pl.ds(base, chunk)],
                                ibuf.at[pl.ds(0, chunk)])

            @pl.when(w == _NW - 1)
            def _():
                pltpu.sync_copy(src_hbm.at[pl.ds(base, last)], ibuf)

            n16 = jnp.where(w == _NW - 1, last // 16, chunk // 16)

            def body(i, c):
                v = ibuf[pl.ds(i * 16, 16)]
                idx = obase * 16 + (v >> _SH) * 16 + lanes
                plsc.addupdate_scatter(hist, [idx], ones)
                return c
            lax.fori_loop(0, n16, body, 0)

        do_list(pd, 0)
        do_list(nd, 0)
        do_list(ps, nbd)
        do_list(ns, nbd + nbv)
        pltpu.sync_copy(hist, cnt_out.at[pl.ds(w * lanes_hist, lanes_hist)])

    return k(pos_src, pos_dst, neg_src, neg_dst)


def _sc_bucket(pos_src, pos_dst, neg_src, neg_dst, nbd, nbv, Nv):
    E = pos_src.shape[0]
    chunk = (E // _NW) // 16 * 16
    last = E - (_NW - 1) * chunk
    capd = nbd * _NW * _SEG_D + 2 * _CH
    capv = nbv * _NW * _SEG_V + 2 * _CH
    i32 = jnp.int32

    @functools.partial(
        pl.kernel, mesh=plsc.VectorSubcoreMesh(**_MESH),
        compiler_params=pltpu.CompilerParams(needs_layout_passes=False),
        out_type=tuple(jax.ShapeDtypeStruct((c,), i32)
                       for c in (capd, capd, capv, capv, capv, capv)),
        scratch_types=[pltpu.VMEM((last,), i32),
                       pltpu.VMEM((last,), i32),
                       pltpu.VMEM((nbd * _SEG_D,), i32),
                       pltpu.VMEM((nbd * _SEG_D,), i32),
                       pltpu.VMEM((nbv * _SEG_V,), i32),
                       pltpu.VMEM((nbv * _SEG_V,), i32),
                       pltpu.VMEM((-(-nbd // 16) * 16,), i32),
                       pltpu.VMEM((-(-nbv // 16) * 16,), i32)])
    def k(ps, pd, ns, nd, DG, DD, PG, PD, NG, ND,
          sbuf, dbuf, dgb, ddb, vgb, vdb, curd, curv):
        w = _wid()
        base = w * chunk
        n16 = jnp.where(w == _NW - 1, last // 16, chunk // 16)
        lanes = lax.iota(jnp.int32, 16)

        def stage(src_hbm, dst_hbm):
            @pl.when(w < _NW - 1)
            def _():
                pltpu.sync_copy(src_hbm.at[pl.ds(base, chunk)],
                                sbuf.at[pl.ds(0, chunk)])
                pltpu.sync_copy(dst_hbm.at[pl.ds(base, chunk)],
                                dbuf.at[pl.ds(0, chunk)])

            @pl.when(w == _NW - 1)
            def _():
                pltpu.sync_copy(src_hbm.at[pl.ds(base, last)], sbuf)
                pltpu.sync_copy(dst_hbm.at[pl.ds(base, last)], dbuf)

        def init_cur(cur, seg):
            def ic(i, c):
                cur[pl.ds(i * 16, 16)] = (i * 16 + lanes) * seg
                return c
            lax.fori_loop(0, cur.shape[0] // 16, ic, 0)

        def edges(goff, c):
            def body(i, cc):
                s = sbuf[pl.ds(i * 16, 16)]
                d = dbuf[pl.ds(i * 16, 16)]
                bd = d >> _SH
                cnt, is_last = plsc.scan_count(bd)
                p = plsc.load_gather(curd, [bd]) + cnt - 1
                plsc.store_scatter(dgb, [p], s + goff)
                plsc.store_scatter(ddb, [p], d)
                plsc.store_scatter(curd, [bd], p + 1, mask=is_last)
                bv = s >> _SH
                vcnt, vlast = plsc.scan_count(bv)
                q = plsc.load_gather(curv, [bv]) + vcnt - 1
                plsc.store_scatter(vgb, [q], d)
                plsc.store_scatter(vdb, [q], s)
                plsc.store_scatter(curv, [bv], q + 1, mask=vlast)
                return cc
            lax.fori_loop(0, n16, body, c)

        def flushv(G, D):
            def fl(b, c):
                off = (b * _NW + w) * _SEG_V
                pltpu.sync_copy(vgb.at[pl.ds(b * _SEG_V, _SEG_V)],
                                G.at[pl.ds(off, _SEG_V)])
                pltpu.sync_copy(vdb.at[pl.ds(b * _SEG_V, _SEG_V)],
                                D.at[pl.ds(off, _SEG_V)])
                return c
            lax.fori_loop(0, nbv, fl, 0)

        init_cur(curd, _SEG_D)
        init_cur(curv, _SEG_V)
        stage(ps, pd)
        edges(0, 0)
        flushv(PG, PD)
        init_cur(curv, _SEG_V)
        stage(ns, nd)
        edges(Nv, 0)
        flushv(NG, ND)

        def fld(b, c):
            off = (b * _NW + w) * _SEG_D
            pltpu.sync_copy(dgb.at[pl.ds(b * _SEG_D, _SEG_D)],
                            DG.at[pl.ds(off, _SEG_D)])
            pltpu.sync_copy(ddb.at[pl.ds(b * _SEG_D, _SEG_D)],
                            DD.at[pl.ds(off, _SEG_D)])
            return c
        lax.fori_loop(0, nbd, fld, 0)

    return k(pos_src, pos_dst, neg_src, neg_dst)


def _zero_acc(zb, accs, sid):
    q = _CB // 16
    for acc in accs:
        o = 0
        while o < q:
            s = min(zb.shape[0], q - o)
            pltpu.sync_copy(zb.at[pl.ds(0, s)], acc.at[pl.ds(sid * q + o, s)])
            o += s


def _flush_acc(pairs, b, rows, sid):
    q = -(-rows // 128) * 8
    full = rows // q
    rem = rows - full * q

    @pl.when(sid < full)
    def _():
        for acc, out in pairs:
            pltpu.sync_copy(acc.at[pl.ds(sid * q, q)],
                            out.at[pl.ds(b * _CB + sid * q, q)])

    if rem:
        @pl.when(sid == full)
        def _():
            for acc, out in pairs:
                pltpu.sync_copy(acc.at[pl.ds(full * q, rem)],
                                out.at[pl.ds(b * _CB + full * q, rem)])


def _zero_zb(zb, H):
    zeros = jnp.zeros((16,), jnp.float32)

    def zr(i, c):
        def zc(j, cc):
            zb[i, pl.ds(j * 16, 16)] = zeros
            return cc
        return lax.fori_loop(0, H // 16, zc, c)
    lax.fori_loop(0, zb.shape[0], zr, 0)


def _consume_pipe(idxg, idxd, cp1, cp2, seg0, segcap, b, pairs,
                  gst, dst_, ig2, id2, semg, sema, cntv):
    lanes = lax.iota(jnp.int32, 16)
    n1 = jnp.sum(cntv[pl.ds(cp1, 16)])
    n2 = jnp.sum(cntv[pl.ds(cp2, 16)])
    c1 = (n1 + _CH - 1) // _CH
    c2 = (n2 + _CH - 1) // _CH
    nch = c1 + c2
    pltpu.sync_copy(idxg.at[pl.ds(seg0, 2 * segcap)],
                    gst.at[pl.ds(0, 2 * segcap)])
    pltpu.sync_copy(idxd.at[pl.ds(seg0, 2 * segcap)],
                    dst_.at[pl.ds(0, 2 * segcap)])

    vpc = _CH // 16

    def clean(soff, nvalid, kbase):
        def body(i, c):
            pos = i * 16
            m = (lanes + pos) < nvalid
            g = gst[pl.ds(soff + pos, 16)]
            d = dst_[pl.ds(soff + pos, 16)]
            kj = kbase * vpc + i
            r = kj // vpc
            cc = (kj % vpc) * 16
            ig2[r, pl.ds(cc, 16)] = jnp.where(m, g, lanes)
            id2[r, pl.ds(cc, 16)] = jnp.where(m, d - b * _CB, _CB)
            return c
        return body

    lax.fori_loop(0, c1 * vpc, clean(0, n1, 0), 0)
    lax.fori_loop(0, c2 * vpc, clean(segcap, n2, c1), 0)

    def start_gathers(k, p):
        for tab, acc, rbs in pairs:
            pltpu.async_copy(tab.at[ig2.at[k]], rbs[p], semg[p])

    def wait_gathers_start_adds(k, p):
        for tab, acc, rbs in pairs:
            pltpu.make_async_copy(tab.at[ig2.at[k]], rbs[p], semg[p]).wait()
        for tab, acc, rbs in pairs:
            pltpu.async_copy(rbs[p], acc.at[id2.at[k]], sema[p], add=True)

    def wait_adds(k, p):
        for tab, acc, rbs in pairs:
            pltpu.make_async_copy(rbs[p], acc.at[id2.at[k]], sema[p]).wait()

    def pair_body(k2, c):
        k0 = 2 * k2

        @pl.when(k0 < nch)
        def _():
            @pl.when(k2 > 0)
            def _():
                wait_adds(k0 - 2, 0)
            start_gathers(k0, 0)

            @pl.when(k2 > 0)
            def _():
                wait_gathers_start_adds(k0 - 1, 1)

        @pl.when(k0 + 1 < nch)
        def _():
            @pl.when(k2 > 0)
            def _():
                wait_adds(k0 - 1, 1)
            start_gathers(k0 + 1, 1)
            wait_gathers_start_adds(k0, 0)
        return c

    lax.fori_loop(0, (nch + 1) // 2, pair_body, 0)

    @pl.when(nch % 2 == 1)
    def _():
        wait_gathers_start_adds(nch - 1, 0)

        @pl.when(nch > 1)
        def _():
            wait_adds(nch - 2, 1)
        wait_adds(nch - 1, 0)

    @pl.when(jnp.logical_and(nch > 0, nch % 2 == 0))
    def _():
        wait_gathers_start_adds(nch - 1, 1)
        wait_adds(nch - 2, 0)
        wait_adds(nch - 1, 1)


def _sc_seg(ltab, qtab, ctab, DG, DD, PG, PD, NG, ND, cnt, nbd, nbv,
            Nc, Nv):
    H = ltab.shape[1]
    nh = nbd + 2 * nbv
    f32 = jnp.float32
    i32 = jnp.int32

    @functools.partial(
        pl.kernel, mesh=plsc.VectorSubcoreMesh(**_MESH),
        compiler_params=pltpu.CompilerParams(needs_layout_passes=False),
        out_type=(jax.ShapeDtypeStruct((Nc, H), f32),
                  jax.ShapeDtypeStruct((Nc, H), f32),
                  jax.ShapeDtypeStruct((Nv, H), f32),
                  jax.ShapeDtypeStruct((Nv, H), f32)),
        scratch_types=[pltpu.VMEM((1216,), i32), pltpu.VMEM((1216,), i32),
                       pltpu.VMEM((18, _CH), i32), pltpu.VMEM((18, _CH), i32),
                       pltpu.VMEM((_CH, H), f32), pltpu.VMEM((_CH, H), f32),
                       pltpu.VMEM((_CH, H), f32), pltpu.VMEM((_CH, H), f32),
                       pltpu.VMEM((40, H), f32),
                       pltpu.VMEM((2 * nh * 16,), i32),
                       pltpu.VMEM_SHARED((_CB + 128, H), f32),
                       pltpu.VMEM_SHARED((_CB + 128, H), f32),
                       pltpu.SemaphoreType.DMA, pltpu.SemaphoreType.DMA,
                       pltpu.SemaphoreType.DMA, pltpu.SemaphoreType.DMA])
    def k(lt, qt, ct, dg, dd, pg, pd, ng, nd, cv, l2c, ea, pc2l, nc2l,
          gst, dst_, ig2, id2, ra0, ra1, rb0, rb1, zb, cntv,
          acca, accb, semg0, semg1, sema0, sema1):
        cid = lax.axis_index("c")
        sid = lax.axis_index("s")
        semg = (semg0, semg1)
        sema = (sema0, sema1)
        hl = nh * 16
        for tt in range(2):
            t = sid * 2 + tt
            pltpu.sync_copy(cv.at[pl.ds(t * hl, hl)],
                            cntv.at[pl.ds(tt * hl, hl)])
        _zero_zb(zb, H)
        _zero_acc(zb, [acca, accb], sid)
        plsc.subcore_barrier()

        def block_c(b, rows):
            seg0 = (b * _NW + 2 * sid) * _SEG_D
            _consume_pipe(dg, dd, b * 16, (nh + b) * 16, seg0, _SEG_D, b,
                          ((lt, acca, (ra0, ra1)), (qt, accb, (rb0, rb1))),
                          gst, dst_, ig2, id2, semg, sema, cntv)
            plsc.subcore_barrier()
            _flush_acc([(acca, l2c), (accb, ea)], b, rows, sid)
            if rows != _CB:
                plsc.subcore_barrier()
            _zero_acc(zb, [acca, accb], sid)
            plsc.subcore_barrier()

        def block_v(b, rows):
            seg0 = (b * _NW + 2 * sid) * _SEG_V
            _consume_pipe(pg, pd, (nbd + b) * 16, (nh + nbd + b) * 16,
                          seg0, _SEG_V, b, ((ct, acca, (ra0, ra1)),),
                          gst, dst_, ig2, id2, semg, sema, cntv)
            _consume_pipe(ng, nd, (nbd + nbv + b) * 16,
                          (nh + nbd + nbv + b) * 16,
                          seg0, _SEG_V, b, ((ct, accb, (rb0, rb1)),),
                          gst, dst_, ig2, id2, semg, sema, cntv)
            plsc.subcore_barrier()
            _flush_acc([(acca, pc2l), (accb, nc2l)], b, rows, sid)
            if rows != _CB:
                plsc.subcore_barrier()
            _zero_acc(zb, [acca, accb], sid)
            plsc.subcore_barrier()

        def run_dir(nb, Ntot, blockfn):
            lastb = nb - 1

            def lb(i, c):
                blockfn(cid + 2 * i, _CB)
                return c
            lax.fori_loop(0, (lastb - cid + 1) // 2, lb, 0)

            @pl.when(cid == (lastb % 2))
            def _():
                blockfn(lastb, Ntot - lastb * _CB)

        run_dir(nbd, Nc, block_c)
        run_dir(nbv, Nv, block_v)

    return k(ltab, qtab, ctab, DG, DD, PG, PD, NG, ND, cnt)


def kernel(l_embedding, c_embedding, noise, pos_src, pos_dst, neg_src, neg_dst,
           q_params, v_params, c_params):
    Nv = l_embedding.shape[0] // 2
    Nc = c_embedding.shape[0]
    H = l_embedding.shape[1]
    nbd = -(-Nc // _CB)
    nbv = -(-Nv // _CB)
    cnt = _sc_hist(pos_src, pos_dst, neg_src, neg_dst, nbd, nbv)
    DG, DD, PG, PD, NG, ND = _sc_bucket(pos_src, pos_dst, neg_src, neg_dst,
                                        nbd, nbv, Nv)
    l2 = l_embedding.reshape(2, Nv, H)

    def round_fn(carry, nz):
        l2, c_emb = carry
        q2 = _qmlp(l2, nz, q_params, 1000)
        l2c, e_arg, pos_c2l, neg_c2l = _sc_seg(
            l2.reshape(2 * Nv, H), q2.reshape(2 * Nv, H), c_emb,
            DG, DD, PG, PD, NG, ND, cnt, nbd, nbv, Nc, Nv)
        l2_new = _vmlp(l2, pos_c2l, neg_c2l, v_params, 1000)
        c_new = _cmlp(l2c, c_emb, e_arg, c_params, 2000)
        return (l2_new, c_new), 0

    (l2, c_embedding), _ = lax.scan(round_fn, (l2, c_embedding), noise)
    return (l2.reshape(2 * Nv, H), c_embedding)


# speedup vs baseline: 1.0270x; 1.0270x over previous
import functools

import jax
import jax.numpy as jnp
from jax import lax
from jax.experimental import pallas as pl
from jax.experimental.pallas import tpu as pltpu
from jax.experimental.pallas import tpu_sc as plsc

_CB = 2048
_SH = 11
_SEG_D = 336
_SEG_V = 560
_CH = 64
_NW = 32


def _softplus(x):
    return jnp.maximum(x, 0.0) + jnp.log1p(jnp.exp(-jnp.abs(x)))


def _dot(a, b):
    return jax.lax.dot(a, b, preferred_element_type=jnp.float32)


def _qmlp_body(l2_ref, nz_ref, w0p, w0n, w0z, b0, w1, b1, w2, b2, q2_ref):
    pos = l2_ref[0]
    neg = l2_ref[1]
    h = _dot(pos, w0p[...]) + _dot(neg, w0n[...]) + _dot(nz_ref[...], w0z[...])
    h = jnp.maximum(h + b0[...], 0.0)
    h = jnp.maximum(_dot(h, w1[...]) + b1[...], 0.0)
    q = _dot(h, w2[...]) + b2[...]
    q2_ref[0] = _softplus(q)
    q2_ref[1] = _softplus(-q)


def _qmlp(l2, nz, q_params, T):
    W0, b0, W1, b1, W2, b2 = q_params
    Nv, H = l2.shape[1], l2.shape[2]
    P = nz.shape[1]
    w0p, w0n, w0z = W0[:H], W0[H:2 * H], W0[2 * H:]
    grid = Nv // T
    full = lambda shape: pl.BlockSpec(shape, lambda i: (0,) * len(shape))
    return pl.pallas_call(
        _qmlp_body,
        grid=(grid,),
        in_specs=[
            pl.BlockSpec((2, T, H), lambda i: (0, i, 0)),
            pl.BlockSpec((T, P), lambda i: (i, 0)),
            full(w0p.shape), full(w0n.shape), full(w0z.shape), full(b0.shape),
            full(W1.shape), full(b1.shape), full(W2.shape), full(b2.shape),
        ],
        out_specs=pl.BlockSpec((2, T, H), lambda i: (0, i, 0)),
        out_shape=jax.ShapeDtypeStruct((2, Nv, H), jnp.float32),
    )(l2, nz, w0p, w0n, w0z, b0, W1, b1, W2, b2)


def _vmlp_body(l2_ref, pc_ref, nc_ref, w0a, w0b, w0c, w0d, b0, w1, b1,
               w2a, w2b, b2a, b2b, o2_ref):
    h = (_dot(l2_ref[0], w0a[...]) + _dot(l2_ref[1], w0b[...])
         + _dot(pc_ref[...], w0c[...]) + _dot(nc_ref[...], w0d[...]))
    h = jnp.maximum(h + b0[...], 0.0)
    h = jnp.maximum(_dot(h, w1[...]) + b1[...], 0.0)
    o2_ref[0] = _dot(h, w2a[...]) + b2a[...]
    o2_ref[1] = _dot(h, w2b[...]) + b2b[...]


def _vmlp(l2, pc2l, nc2l, v_params, T):
    W0, b0, W1, b1, W2, b2 = v_params
    Nv, H = l2.shape[1], l2.shape[2]
    w0a, w0b, w0c, w0d = W0[:H], W0[H:2 * H], W0[2 * H:3 * H], W0[3 * H:]
    w2a, w2b = W2[:, :H], W2[:, H:]
    b2a, b2b = b2[:H], b2[H:]
    grid = Nv // T
    full = lambda shape: pl.BlockSpec(shape, lambda i: (0,) * len(shape))
    return pl.pallas_call(
        _vmlp_body,
        grid=(grid,),
        in_specs=[
            pl.BlockSpec((2, T, H), lambda i: (0, i, 0)),
            pl.BlockSpec((T, H), lambda i: (i, 0)),
            pl.BlockSpec((T, H), lambda i: (i, 0)),
            full(w0a.shape), full(w0b.shape), full(w0c.shape), full(w0d.shape),
            full(b0.shape), full(W1.shape), full(b1.shape),
            full(w2a.shape), full(w2b.shape), full(b2a.shape), full(b2b.shape),
        ],
        out_specs=pl.BlockSpec((2, T, H), lambda i: (0, i, 0)),
        out_shape=jax.ShapeDtypeStruct((2, Nv, H), jnp.float32),
    )(l2, pc2l, nc2l, w0a, w0b, w0c, w0d, b0, W1, b1, w2a, w2b, b2a, b2b)


def _cmlp_body(l2c_ref, ce_ref, ea_ref, w0a, w0b, w0c, b0, w1, b1, w2, b2,
               out_ref):
    e = jnp.exp(-ea_ref[...])
    h = (_dot(l2c_ref[...], w0a[...]) + _dot(ce_ref[...], w0b[...])
         + _dot(e, w0c[...]))
    h = jnp.maximum(h + b0[...], 0.0)
    h = jnp.maximum(_dot(h, w1[...]) + b1[...], 0.0)
    out_ref[...] = _dot(h, w2[...]) + b2[...]


def _cmlp(l2c, c_emb, e_arg, c_params, T):
    W0, b0, W1, b1, W2, b2 = c_params
    Nc, H = c_emb.shape
    w0a, w0b, w0c = W0[:H], W0[H:2 * H], W0[2 * H:]
    grid = Nc // T
    full = lambda shape: pl.BlockSpec(shape, lambda i: (0,) * len(shape))
    row = pl.BlockSpec((T, H), lambda i: (i, 0))
    return pl.pallas_call(
        _cmlp_body,
        grid=(grid,),
        in_specs=[row, row, row,
                  full(w0a.shape), full(w0b.shape), full(w0c.shape),
                  full(b0.shape), full(W1.shape), full(b1.shape),
                  full(W2.shape), full(b2.shape)],
        out_specs=row,
        out_shape=jax.ShapeDtypeStruct((Nc, H), jnp.float32),
    )(l2c, c_emb, e_arg, w0a, w0b, w0c, b0, W1, b1, W2, b2)


_MESH = dict(core_axis_name="c", subcore_axis_name="s")


def _wid():
    return lax.axis_index("s") * 2 + lax.axis_index("c")


def _sc_hist(pos_src, pos_dst, neg_src, neg_dst, nbd, nbv):
    E = pos_src.shape[0]
    chunk = (E // _NW) // 16 * 16
    last = E - (_NW - 1) * chunk
    nh = nbd + 2 * nbv
    lanes_hist = nh * 16

    @functools.partial(
        pl.kernel, mesh=plsc.VectorSubcoreMesh(**_MESH),
        compiler_params=pltpu.CompilerParams(needs_layout_passes=False),
        out_type=jax.ShapeDtypeStruct((_NW * lanes_hist,), jnp.int32),
        scratch_types=[pltpu.VMEM((last,), jnp.int32),
                       pltpu.VMEM((lanes_hist,), jnp.int32)])
    def k(ps, pd, ns, nd, cnt_out, ibuf, hist):
        w = _wid()
        base = w * chunk
        lanes = lax.iota(jnp.int32, 16)
        ones = jnp.ones((16,), jnp.int32)

        def zb(i, c):
            hist[pl.ds(i * 16, 16)] = jnp.zeros((16,), jnp.int32)
            return c
        lax.fori_loop(0, nh, zb, 0)

        def do_list(src_hbm, obase):
            @pl.when(w < _NW - 1)
            def _():
                pltpu.sync_copy(src_hbm.at[pl.ds(base, chunk)],
                                ibuf.at[pl.ds(0, chunk)])

            @pl.when(w == _NW - 1)
            def _():
                pltpu.sync_copy(src_hbm.at[pl.ds(base, last)], ibuf)

            n16 = jnp.where(w == _NW - 1, last // 16, chunk // 16)

            def body(i, c):
                v = ibuf[pl.ds(i * 16, 16)]
                idx = obase * 16 + (v >> _SH) * 16 + lanes
                plsc.addupdate_scatter(hist, [idx], ones)
                return c
            lax.fori_loop(0, n16, body, 0)

        do_list(pd, 0)
        do_list(nd, 0)
        do_list(ps, nbd)
        do_list(ns, nbd + nbv)
        pltpu.sync_copy(hist, cnt_out.at[pl.ds(w * lanes_hist, lanes_hist)])

    return k(pos_src, pos_dst, neg_src, neg_dst)


def _sc_bucket(pos_src, pos_dst, neg_src, neg_dst, nbd, nbv, Nv):
    E = pos_src.shape[0]
    chunk = (E // _NW) // 16 * 16
    last = E - (_NW - 1) * chunk
    capd = nbd * _NW * _SEG_D + 2 * _CH
    capv = nbv * _NW * _SEG_V + 2 * _CH
    i32 = jnp.int32

    @functools.partial(
        pl.kernel, mesh=plsc.VectorSubcoreMesh(**_MESH),
        compiler_params=pltpu.CompilerParams(needs_layout_passes=False),
        out_type=tuple(jax.ShapeDtypeStruct((c,), i32)
                       for c in (capd, capd, capv, capv, capv, capv)),
        scratch_types=[pltpu.VMEM((last,), i32),
                       pltpu.VMEM((last,), i32),
                       pltpu.VMEM((nbd * _SEG_D,), i32),
                       pltpu.VMEM((nbd * _SEG_D,), i32),
                       pltpu.VMEM((nbv * _SEG_V,), i32),
                       pltpu.VMEM((nbv * _SEG_V,), i32),
                       pltpu.VMEM((-(-nbd // 16) * 16,), i32),
                       pltpu.VMEM((-(-nbv // 16) * 16,), i32)])
    def k(ps, pd, ns, nd, DG, DD, PG, PD, NG, ND,
          sbuf, dbuf, dgb, ddb, vgb, vdb, curd, curv):
        w = _wid()
        base = w * chunk
        n16 = jnp.where(w == _NW - 1, last // 16, chunk // 16)
        lanes = lax.iota(jnp.int32, 16)

        def stage(src_hbm, dst_hbm):
            @pl.when(w < _NW - 1)
            def _():
                pltpu.sync_copy(src_hbm.at[pl.ds(base, chunk)],
                                sbuf.at[pl.ds(0, chunk)])
                pltpu.sync_copy(dst_hbm.at[pl.ds(base, chunk)],
                                dbuf.at[pl.ds(0, chunk)])

            @pl.when(w == _NW - 1)
            def _():
                pltpu.sync_copy(src_hbm.at[pl.ds(base, last)], sbuf)
                pltpu.sync_copy(dst_hbm.at[pl.ds(base, last)], dbuf)

        def init_cur(cur, seg):
            def ic(i, c):
                cur[pl.ds(i * 16, 16)] = (i * 16 + lanes) * seg
                return c
            lax.fori_loop(0, cur.shape[0] // 16, ic, 0)

        def edges(goff, c):
            def body(i, cc):
                s = sbuf[pl.ds(i * 16, 16)]
                d = dbuf[pl.ds(i * 16, 16)]
                bd = d >> _SH
                cnt, is_last = plsc.scan_count(bd)
                p = plsc.load_gather(curd, [bd]) + cnt - 1
                plsc.store_scatter(dgb, [p], s + goff)
                plsc.store_scatter(ddb, [p], d)
                plsc.store_scatter(curd, [bd], p + 1, mask=is_last)
                bv = s >> _SH
                vcnt, vlast = plsc.scan_count(bv)
                q = plsc.load_gather(curv, [bv]) + vcnt - 1
                plsc.store_scatter(vgb, [q], d)
                plsc.store_scatter(vdb, [q], s)
                plsc.store_scatter(curv, [bv], q + 1, mask=vlast)
                return cc
            lax.fori_loop(0, n16, body, c)

        def flushv(G, D):
            def fl(b, c):
                off = (b * _NW + w) * _SEG_V
                pltpu.sync_copy(vgb.at[pl.ds(b * _SEG_V, _SEG_V)],
                                G.at[pl.ds(off, _SEG_V)])
                pltpu.sync_copy(vdb.at[pl.ds(b * _SEG_V, _SEG_V)],
                                D.at[pl.ds(off, _SEG_V)])
                return c
            lax.fori_loop(0, nbv, fl, 0)

        init_cur(curd, _SEG_D)
        init_cur(curv, _SEG_V)
        stage(ps, pd)
        edges(0, 0)
        flushv(PG, PD)
        init_cur(curv, _SEG_V)
        stage(ns, nd)
        edges(Nv, 0)
        flushv(NG, ND)

        def fld(b, c):
            off = (b * _NW + w) * _SEG_D
            pltpu.sync_copy(dgb.at[pl.ds(b * _SEG_D, _SEG_D)],
                            DG.at[pl.ds(off, _SEG_D)])
            pltpu.sync_copy(ddb.at[pl.ds(b * _SEG_D, _SEG_D)],
                            DD.at[pl.ds(off, _SEG_D)])
            return c
        lax.fori_loop(0, nbd, fld, 0)

    return k(pos_src, pos_dst, neg_src, neg_dst)


def _zero_acc(zb, accs, sid):
    q = _CB // 16
    for acc in accs:
        o = 0
        while o < q:
            s = min(zb.shape[0], q - o)
            pltpu.sync_copy(zb.at[pl.ds(0, s)], acc.at[pl.ds(sid * q + o, s)])
            o += s


def _flush_acc(pairs, b, rows, sid):
    q = -(-rows // 128) * 8
    full = rows // q
    rem = rows - full * q

    @pl.when(sid < full)
    def _():
        for acc, out in pairs:
            pltpu.sync_copy(acc.at[pl.ds(sid * q, q)],
                            out.at[pl.ds(b * _CB + sid * q, q)])

    if rem:
        @pl.when(sid == full)
        def _():
            for acc, out in pairs:
                pltpu.sync_copy(acc.at[pl.ds(full * q, rem)],
                                out.at[pl.ds(b * _CB + full * q, rem)])


def _zero_zb(zb, H):
    zeros = jnp.zeros((16,), jnp.float32)

    def zr(i, c):
        def zc(j, cc):
            zb[i, pl.ds(j * 16, 16)] = zeros
            return cc
        return lax.fori_loop(0, H // 16, zc, c)
    lax.fori_loop(0, zb.shape[0], zr, 0)


def _consume_pipe(idxg, idxd, cp1, cp2, seg0, segcap, b, pairs,
                  gst, dst_, ig2, id2, semg, sema, cntv):
    lanes = lax.iota(jnp.int32, 16)
    n1 = jnp.sum(cntv[pl.ds(cp1, 16)])
    n2 = jnp.sum(cntv[pl.ds(cp2, 16)])
    c1 = (n1 + _CH - 1) // _CH
    c2 = (n2 + _CH - 1) // _CH
    nch = c1 + c2
    pltpu.sync_copy(idxg.at[pl.ds(seg0, 2 * segcap)],
                    gst.at[pl.ds(0, 2 * segcap)])
    pltpu.sync_copy(idxd.at[pl.ds(seg0, 2 * segcap)],
                    dst_.at[pl.ds(0, 2 * segcap)])

    vpc = _CH // 16

    def clean(soff, nvalid, kbase):
        def body(i, c):
            pos = i * 16
            m = (lanes + pos) < nvalid
            g = gst[pl.ds(soff + pos, 16)]
            d = dst_[pl.ds(soff + pos, 16)]
            kj = kbase * vpc + i
            r = kj // vpc
            cc = (kj % vpc) * 16
            ig2[r, pl.ds(cc, 16)] = jnp.where(m, g, lanes)
            id2[r, pl.ds(cc, 16)] = jnp.where(m, d - b * _CB, _CB)
            return c
        return body

    lax.fori_loop(0, c1 * vpc, clean(0, n1, 0), 0)
    lax.fori_loop(0, c2 * vpc, clean(segcap, n2, c1), 0)

    def start_gathers(k, p):
        for tab, acc, rbs in pairs:
            pltpu.async_copy(tab.at[ig2.at[k]], rbs[p], semg[p])

    def wait_gathers_start_adds(k, p):
        for tab, acc, rbs in pairs:
            pltpu.make_async_copy(tab.at[ig2.at[k]], rbs[p], semg[p]).wait()
        for tab, acc, rbs in pairs:
            pltpu.async_copy(rbs[p], acc.at[id2.at[k]], sema[p], add=True)

    def wait_adds(k, p):
        for tab, acc, rbs in pairs:
            pltpu.make_async_copy(rbs[p], acc.at[id2.at[k]], sema[p]).wait()

    def pair_body(k2, c):
        k0 = 2 * k2

        @pl.when(k0 < nch)
        def _():
            @pl.when(k2 > 0)
            def _():
                wait_adds(k0 - 2, 0)
            start_gathers(k0, 0)

            @pl.when(k2 > 0)
            def _():
                wait_gathers_start_adds(k0 - 1, 1)

        @pl.when(k0 + 1 < nch)
        def _():
            @pl.when(k2 > 0)
            def _():
                wait_adds(k0 - 1, 1)
            start_gathers(k0 + 1, 1)
            wait_gathers_start_adds(k0, 0)
        return c

    lax.fori_loop(0, (nch + 1) // 2, pair_body, 0)

    @pl.when(nch % 2 == 1)
    def _():
        wait_gathers_start_adds(nch - 1, 0)

        @pl.when(nch > 1)
        def _():
            wait_adds(nch - 2, 1)
        wait_adds(nch - 1, 0)

    @pl.when(jnp.logical_and(nch > 0, nch % 2 == 0))
    def _():
        wait_gathers_start_adds(nch - 1, 1)
        wait_adds(nch - 2, 0)
        wait_adds(nch - 1, 1)


def _sc_dir(tabs, idxs, cnt, nb, cps, Ntot, H, nh, two_tables):
    f32 = jnp.float32
    i32 = jnp.int32
    segcap = _SEG_D if two_tables else _SEG_V

    @functools.partial(
        pl.kernel, mesh=plsc.VectorSubcoreMesh(**_MESH),
        compiler_params=pltpu.CompilerParams(needs_layout_passes=False),
        out_type=(jax.ShapeDtypeStruct((Ntot, H), f32),
                  jax.ShapeDtypeStruct((Ntot, H), f32)),
        scratch_types=[pltpu.VMEM((1216,), i32), pltpu.VMEM((1216,), i32),
                       pltpu.VMEM((18, _CH), i32), pltpu.VMEM((18, _CH), i32),
                       pltpu.VMEM((_CH, H), f32), pltpu.VMEM((_CH, H), f32),
                       pltpu.VMEM((_CH, H), f32), pltpu.VMEM((_CH, H), f32),
                       pltpu.VMEM((40, H), f32),
                       pltpu.VMEM((2 * nh * 16,), i32),
                       pltpu.VMEM_SHARED((_CB + 128, H), f32),
                       pltpu.VMEM_SHARED((_CB + 128, H), f32),
                       pltpu.SemaphoreType.DMA, pltpu.SemaphoreType.DMA,
                       pltpu.SemaphoreType.DMA, pltpu.SemaphoreType.DMA])
    def k(*refs):
        nt = len(tabs)
        ni = len(idxs)
        tr = refs[:nt]
        ir = refs[nt:nt + ni]
        cv = refs[nt + ni]
        o1, o2 = refs[nt + ni + 1:nt + ni + 3]
        (gst, dst_, ig2, id2, ra0, ra1, rb0, rb1, zb, cntv,
         acca, accb, semg0, semg1, sema0, sema1) = refs[nt + ni + 3:]
        cid = lax.axis_index("c")
        sid = lax.axis_index("s")
        semg = (semg0, semg1)
        sema = (sema0, sema1)
        hl = nh * 16
        for tt in range(2):
            t = sid * 2 + tt
            pltpu.sync_copy(cv.at[pl.ds(t * hl, hl)],
                            cntv.at[pl.ds(tt * hl, hl)])
        _zero_zb(zb, H)
        _zero_acc(zb, [acca, accb], sid)
        plsc.subcore_barrier()

        def blockfn(b, rows):
            seg0 = (b * _NW + 2 * sid) * segcap
            if two_tables:
                _consume_pipe(ir[0], ir[1], (cps[0] + b) * 16,
                              (nh + cps[0] + b) * 16, seg0, segcap, b,
                              ((tr[0], acca, (ra0, ra1)),
                               (tr[1], accb, (rb0, rb1))),
                              gst, dst_, ig2, id2, semg, sema, cntv)
            else:
                _consume_pipe(ir[0], ir[1], (cps[0] + b) * 16,
                              (nh + cps[0] + b) * 16, seg0, segcap, b,
                              ((tr[0], acca, (ra0, ra1)),),
                              gst, dst_, ig2, id2, semg, sema, cntv)
                _consume_pipe(ir[2], ir[3], (cps[1] + b) * 16,
                              (nh + cps[1] + b) * 16, seg0, segcap, b,
                              ((tr[0], accb, (rb0, rb1)),),
                              gst, dst_, ig2, id2, semg, sema, cntv)
            plsc.subcore_barrier()
            _flush_acc([(acca, o1), (accb, o2)], b, rows, sid)
            if rows != _CB:
                plsc.subcore_barrier()
            _zero_acc(zb, [acca, accb], sid)
            plsc.subcore_barrier()

        lastb = nb - 1

        def lb(i, c):
            blockfn(cid + 2 * i, _CB)
            return c
        lax.fori_loop(0, (lastb - cid + 1) // 2, lb, 0)

        @pl.when(cid == (lastb % 2))
        def _():
            blockfn(lastb, Ntot - lastb * _CB)

    return k(*tabs, *idxs, cnt)


def kernel(l_embedding, c_embedding, noise, pos_src, pos_dst, neg_src, neg_dst,
           q_params, v_params, c_params):
    Nv = l_embedding.shape[0] // 2
    Nc = c_embedding.shape[0]
    H = l_embedding.shape[1]
    nbd = -(-Nc // _CB)
    nbv = -(-Nv // _CB)
    cnt = _sc_hist(pos_src, pos_dst, neg_src, neg_dst, nbd, nbv)
    DG, DD, PG, PD, NG, ND = _sc_bucket(pos_src, pos_dst, neg_src, neg_dst,
                                        nbd, nbv, Nv)
    l2 = l_embedding.reshape(2, Nv, H)

    nh = nbd + 2 * nbv

    def round_fn(carry, nz):
        l2, c_emb = carry
        q2 = _qmlp(l2, nz, q_params, 1000)
        l2c, e_arg = _sc_dir(
            (l2.reshape(2 * Nv, H), q2.reshape(2 * Nv, H)), (DG, DD),
            cnt, nbd, (0,), Nc, H, nh, True)
        pos_c2l, neg_c2l = _sc_dir(
            (c_emb,), (PG, PD, NG, ND),
            cnt, nbv, (nbd, nbd + nbv), Nv, H, nh, False)
        c_new = _cmlp(l2c, c_emb, e_arg, c_params, 2000)
        l2_new = _vmlp(l2, pos_c2l, neg_c2l, v_params, 1000)
        return (l2_new, c_new), 0

    (l2, c_embedding), _ = lax.scan(round_fn, (l2, c_embedding), noise)
    return (l2.reshape(2 * Nv, H), c_embedding)
